# Initial kernel scaffold; baseline (speedup 1.0000x reference)
import jax
import jax.numpy as jnp
from jax.experimental import pallas as pl


def kernel(x, edge_index, batch, W1, b1, W2, b2, W3, b3, W4, b4, W5, b5, Wc, bc):
    raise NotImplementedError("write your pallas kernel here")


# speedup vs baseline: 13.4403x; 13.4403x over previous
import functools

import jax
import jax.numpy as jnp
from jax import lax
from jax.experimental import pallas as pl
from jax.experimental.pallas import tpu as pltpu
from jax.experimental.pallas import tpu_sc as plsc

_N = 10000
_E = 320000
_G = 64
_F = 128
_NC = 2
_NS = 16
_C = 80
_PER_TILE = _E // (_NC * _NS)
_NCHUNK = _PER_TILE // _C
_RPT = 624
_TAIL0 = _NS * _RPT
_TAIL = _N - _TAIL0

_HIGH = jax.lax.Precision.HIGHEST


def _fill(buf, rows, value):
    @pl.loop(0, rows)
    def _(r):
        @pl.loop(0, _F, step=16)
        def _(c0):
            buf[r, pl.ds(c0, 16)] = jnp.full((16,), value, jnp.float32)


_ZR = 16


def _zero_acc(acc, zbuf, sid):
    row0 = sid * _RPT
    off = 0
    while off < _RPT:
        nr = min(_ZR, _RPT - off)
        pltpu.sync_copy(zbuf.at[pl.ds(0, nr)], acc.at[pl.ds(row0 + off, nr)])
        off += nr

    @pl.when(sid == 0)
    def _():
        pltpu.sync_copy(zbuf.at[pl.ds(0, _TAIL)], acc.at[pl.ds(_TAIL0, _TAIL)])

    return row0


def _copy_out(acc, out_hbm, cid, sid, row0):
    pltpu.sync_copy(acc.at[pl.ds(row0, _RPT)],
                    out_hbm.at[cid, pl.ds(row0, _RPT)])

    @pl.when(sid == 0)
    def _():
        pltpu.sync_copy(acc.at[pl.ds(_TAIL0, _TAIL)],
                        out_hbm.at[cid, pl.ds(_TAIL0, _TAIL)])


def _make_agg():
    mesh = plsc.VectorSubcoreMesh(core_axis_name="c", subcore_axis_name="s")

    @functools.partial(
        pl.kernel, mesh=mesh,
        out_type=jax.ShapeDtypeStruct((_NC, _N, _F), jnp.float32),
        scratch_types=[
            pltpu.VMEM((_NCHUNK, _C), jnp.int32),
            pltpu.VMEM((_NCHUNK, _C), jnp.int32),
            pltpu.VMEM((_C, _F), jnp.float32),
            pltpu.VMEM((_ZR, _F), jnp.float32),
            pltpu.VMEM_SHARED((_N, _F), jnp.float32),
        ],
    )
    def k(p_hbm, src_hbm, dst_hbm, out_hbm, src_v, dst_v, buf, zbuf, acc):
        cid = lax.axis_index("c")
        sid = lax.axis_index("s")
        tile = cid * _NS + sid
        pltpu.sync_copy(src_hbm.at[tile], src_v)
        pltpu.sync_copy(dst_hbm.at[tile], dst_v)
        _fill(zbuf, _ZR, 0.0)
        row0 = _zero_acc(acc, zbuf, sid)
        plsc.subcore_barrier()

        @pl.loop(0, _NCHUNK)
        def _(i):
            pltpu.sync_copy(p_hbm.at[src_v.at[i]], buf)
            pltpu.sync_copy(buf, acc.at[dst_v.at[i]], add=True)

        plsc.subcore_barrier()
        _copy_out(acc, out_hbm, cid, sid, row0)

    return k


def _make_deg():
    mesh = plsc.VectorSubcoreMesh(core_axis_name="c", subcore_axis_name="s")

    @functools.partial(
        pl.kernel, mesh=mesh,
        out_type=jax.ShapeDtypeStruct((_NC, _N, _F), jnp.float32),
        scratch_types=[
            pltpu.VMEM((_NCHUNK, _C), jnp.int32),
            pltpu.VMEM((_C, _F), jnp.float32),
            pltpu.VMEM((_ZR, _F), jnp.float32),
            pltpu.VMEM_SHARED((_N, _F), jnp.float32),
        ],
    )
    def k(dst_hbm, out_hbm, dst_v, ones_v, zbuf, acc):
        cid = lax.axis_index("c")
        sid = lax.axis_index("s")
        tile = cid * _NS + sid
        pltpu.sync_copy(dst_hbm.at[tile], dst_v)
        _fill(ones_v, _C, 1.0)
        _fill(zbuf, _ZR, 0.0)
        row0 = _zero_acc(acc, zbuf, sid)
        plsc.subcore_barrier()

        @pl.loop(0, _NCHUNK)
        def _(i):
            pltpu.sync_copy(ones_v, acc.at[dst_v.at[i]], add=True)

        plsc.subcore_barrier()
        _copy_out(acc, out_hbm, cid, sid, row0)

    return k


_deg_call = _make_deg()
_agg_call = _make_agg()


def _dot(a, b):
    return jnp.dot(a, b, preferred_element_type=jnp.float32, precision=_HIGH)


def _pad(v):
    n, f = v.shape
    if f == _F:
        return v
    return jnp.concatenate([v, jnp.zeros((n, _F - f), jnp.float32)], axis=1)


def _t_xw1(x_ref, w_ref, o_ref):
    o_ref[...] = _pad(_dot(x_ref[...], w_ref[...]))


def _t_dinv_p1(degp_ref, z_ref, dinv_ref, p_ref):
    d = degp_ref[...]
    deg = d[0, :, 0:1] + d[1, :, 0:1] + 1.0
    dinv = jax.lax.rsqrt(deg)
    dinv_ref[...] = dinv
    p_ref[...] = z_ref[...] * dinv


def _t_post1(a_ref, p_ref, dinv_ref, b_ref, o_ref):
    a = a_ref[...]
    dinv = dinv_ref[...]
    s = a[0, :, :16] + a[1, :, :16] + p_ref[...][:, :16]
    h = jnp.tanh(dinv * s + b_ref[...])
    o_ref[...] = _pad(dinv * h)


def _t_post2(a_ref, p_ref, dinv_ref, w_ref, b_ref, o_ref):
    a = a_ref[...]
    dinv = dinv_ref[...]
    t = dinv * (a[0, :, :16] + a[1, :, :16] + p_ref[...][:, :16])
    h = jnp.tanh(_dot(t, w_ref[...]) + b_ref[...])
    o_ref[...] = dinv * h


def _t_post3(a_ref, p_ref, dinv_ref, w3_ref, b3_ref, w4_ref, o_ref):
    a = a_ref[...]
    dinv = dinv_ref[...]
    t = dinv * (a[0] + a[1] + p_ref[...])
    h3 = jnp.tanh(_dot(t, w3_ref[...]) + b3_ref[...])
    o_ref[...] = _pad(dinv * _dot(h3, w4_ref[...]))


def _t_post4(a_ref, p_ref, dinv_ref, b_ref, o_ref):
    a = a_ref[...]
    dinv = dinv_ref[...]
    s = a[0, :, :64] + a[1, :, :64] + p_ref[...][:, :64]
    h = jnp.tanh(dinv * s + b_ref[...])
    o_ref[...] = _pad(dinv * h)


def _t_post5(a_ref, p_ref, dinv_ref, w5_ref, b5_ref, batch_ref, wc_ref,
             bc_ref, o_ref):
    a = a_ref[...]
    dinv = dinv_ref[...]
    t = dinv * (a[0, :, :64] + a[1, :, :64] + p_ref[...][:, :64])
    h5 = jnp.tanh(_dot(t, w5_ref[...]) + b5_ref[...])
    seg = batch_ref[...]
    gid = lax.broadcasted_iota(jnp.int32, (_G, _N), 0)
    oh = (gid == seg).astype(jnp.float32)
    sums = _dot(oh, h5)
    cnts = jnp.sum(oh, axis=1, keepdims=True)
    pooled = sums / jnp.maximum(cnts, 1.0)
    o_ref[...] = _dot(pooled, wc_ref[...]) + bc_ref[...]


def _tc(body, out_shape, *args):
    return pl.pallas_call(body, out_shape=out_shape)(*args)


def kernel(x, edge_index, batch, W1, b1, W2, b2, W3, b3, W4, b4, W5, b5,
           Wc, bc):
    f32 = jnp.float32
    src = edge_index[0].reshape(_NC * _NS, _NCHUNK, _C)
    dst = edge_index[1].reshape(_NC * _NS, _NCHUNK, _C)

    z1 = _tc(_t_xw1, jax.ShapeDtypeStruct((_N, _F), f32), x, W1)
    degp = _deg_call(dst)
    dinv, p1 = _tc(
        _t_dinv_p1,
        [jax.ShapeDtypeStruct((_N, 1), f32), jax.ShapeDtypeStruct((_N, _F), f32)],
        degp, z1)

    a1 = _agg_call(p1, src, dst)
    p2 = _tc(_t_post1, jax.ShapeDtypeStruct((_N, _F), f32),
             a1, p1, dinv, b1.reshape(1, 16))
    a2 = _agg_call(p2, src, dst)
    p3 = _tc(_t_post2, jax.ShapeDtypeStruct((_N, _F), f32),
             a2, p2, dinv, W2, b2.reshape(1, 128))
    a3 = _agg_call(p3, src, dst)
    p4 = _tc(_t_post3, jax.ShapeDtypeStruct((_N, _F), f32),
             a3, p3, dinv, W3, b3.reshape(1, 128), W4)
    a4 = _agg_call(p4, src, dst)
    p5 = _tc(_t_post4, jax.ShapeDtypeStruct((_N, _F), f32),
             a4, p4, dinv, b4.reshape(1, 64))
    a5 = _agg_call(p5, src, dst)
    out = _tc(_t_post5, jax.ShapeDtypeStruct((_G, 1), f32),
              a5, p5, dinv, W5, b5.reshape(1, 64), batch.reshape(1, _N),
              Wc, bc.reshape(1, 1))
    return out


# speedup vs baseline: 20.7477x; 1.5437x over previous
import functools

import jax
import jax.numpy as jnp
from jax import lax
from jax.experimental import pallas as pl
from jax.experimental.pallas import tpu as pltpu
from jax.experimental.pallas import tpu_sc as plsc

_N = 10000
_E = 320000
_G = 64
_F = 128
_NC = 2
_NS = 16
_C = 80
_PER_TILE = _E // (_NC * _NS)
_NCHUNK = _PER_TILE // _C
_RPT = 624
_TAIL0 = _NS * _RPT
_TAIL = _N - _TAIL0

_HIGH = jax.lax.Precision.HIGHEST


def _fill(buf, rows, value):
    @pl.loop(0, rows)
    def _(r):
        @pl.loop(0, _F, step=16)
        def _(c0):
            buf[r, pl.ds(c0, 16)] = jnp.full((16,), value, jnp.float32)


_ZR = 16


def _zero_acc(acc, zbuf, sid):
    row0 = sid * _RPT
    off = 0
    while off < _RPT:
        nr = min(_ZR, _RPT - off)
        pltpu.sync_copy(zbuf.at[pl.ds(0, nr)], acc.at[pl.ds(row0 + off, nr)])
        off += nr

    @pl.when(sid == 0)
    def _():
        pltpu.sync_copy(zbuf.at[pl.ds(0, _TAIL)], acc.at[pl.ds(_TAIL0, _TAIL)])

    return row0


def _copy_out(acc, out_hbm, cid, sid, row0):
    pltpu.sync_copy(acc.at[pl.ds(row0, _RPT)],
                    out_hbm.at[cid, pl.ds(row0, _RPT)])

    @pl.when(sid == 0)
    def _():
        pltpu.sync_copy(acc.at[pl.ds(_TAIL0, _TAIL)],
                        out_hbm.at[cid, pl.ds(_TAIL0, _TAIL)])


def _make_agg():
    mesh = plsc.VectorSubcoreMesh(core_axis_name="c", subcore_axis_name="s")

    @functools.partial(
        pl.kernel, mesh=mesh,
        out_type=jax.ShapeDtypeStruct((_NC, _N, _F), jnp.float32),
        scratch_types=[
            pltpu.VMEM((_PER_TILE,), jnp.int32),
            pltpu.VMEM((_NCHUNK, _C), jnp.int32),
            pltpu.VMEM((_C, _F), jnp.float32),
            pltpu.VMEM((_C, _F), jnp.float32),
            pltpu.VMEM((_ZR, _F), jnp.float32),
            pltpu.VMEM_SHARED((_N, _F), jnp.float32),
            pltpu.SemaphoreType.DMA,
            pltpu.SemaphoreType.DMA,
        ],
    )
    def k(p_hbm, src_hbm, dst_hbm, out_hbm, src_v, dst_v, bufa, bufb, zbuf,
          acc, sema, semb):
        cid = lax.axis_index("c")
        sid = lax.axis_index("s")
        tile = cid * _NS + sid
        pltpu.sync_copy(src_hbm.at[pl.ds(tile * _PER_TILE, _PER_TILE)], src_v)
        pltpu.sync_copy(dst_hbm.at[tile], dst_v)
        _fill(zbuf, _ZR, 0.0)
        row0 = _zero_acc(acc, zbuf, sid)
        plsc.subcore_barrier()

        def issue(i, buf, sem):
            @pl.when(i < _NCHUNK)
            def _():
                pltpu.async_copy(p_hbm.at[src_v.at[pl.ds(i * _C, _C)]],
                                 buf, sem)

        def wait(buf, sem):
            pltpu.make_async_copy(p_hbm.at[src_v.at[pl.ds(0, _C)]],
                                  buf, sem).wait()

        issue(0, bufa, sema)
        issue(1, bufb, semb)

        @pl.loop(0, (_NCHUNK + 1) // 2)
        def _(j):
            i0 = 2 * j
            wait(bufa, sema)
            pltpu.sync_copy(bufa, acc.at[dst_v.at[i0]], add=True)
            issue(i0 + 2, bufa, sema)

            @pl.when(i0 + 1 < _NCHUNK)
            def _():
                wait(bufb, semb)
                pltpu.sync_copy(bufb, acc.at[dst_v.at[i0 + 1]], add=True)
                issue(i0 + 3, bufb, semb)

        plsc.subcore_barrier()
        _copy_out(acc, out_hbm, cid, sid, row0)

    return k


def _make_deg():
    mesh = plsc.VectorSubcoreMesh(core_axis_name="c", subcore_axis_name="s")

    @functools.partial(
        pl.kernel, mesh=mesh,
        out_type=jax.ShapeDtypeStruct((_NC, _N, _F), jnp.float32),
        scratch_types=[
            pltpu.VMEM((_NCHUNK, _C), jnp.int32),
            pltpu.VMEM((_C, _F), jnp.float32),
            pltpu.VMEM((_ZR, _F), jnp.float32),
            pltpu.VMEM_SHARED((_N, _F), jnp.float32),
            pltpu.SemaphoreType.DMA,
        ],
    )
    def k(dst_hbm, out_hbm, dst_v, ones_v, zbuf, acc, sem):
        cid = lax.axis_index("c")
        sid = lax.axis_index("s")
        tile = cid * _NS + sid
        pltpu.sync_copy(dst_hbm.at[tile], dst_v)
        _fill(ones_v, _C, 1.0)
        _fill(zbuf, _ZR, 0.0)
        row0 = _zero_acc(acc, zbuf, sid)
        plsc.subcore_barrier()

        @pl.loop(0, _NCHUNK)
        def _(i):
            pltpu.sync_copy(ones_v, acc.at[dst_v.at[i]], add=True)

        plsc.subcore_barrier()
        _copy_out(acc, out_hbm, cid, sid, row0)

    return k


_deg_call = _make_deg()
_agg_call = _make_agg()


def _dot(a, b):
    return jnp.dot(a, b, preferred_element_type=jnp.float32, precision=_HIGH)


def _pad(v):
    n, f = v.shape
    if f == _F:
        return v
    return jnp.concatenate([v, jnp.zeros((n, _F - f), jnp.float32)], axis=1)


def _t_xw1(x_ref, w_ref, o_ref):
    o_ref[...] = _pad(_dot(x_ref[...], w_ref[...]))


def _t_dinv_p1(degp_ref, z_ref, dinv_ref, p_ref):
    d = degp_ref[...]
    deg = d[0, :, 0:1] + d[1, :, 0:1] + 1.0
    dinv = jax.lax.rsqrt(deg)
    dinv_ref[...] = dinv
    p_ref[...] = z_ref[...] * dinv


def _t_post1(a_ref, p_ref, dinv_ref, b_ref, o_ref):
    a = a_ref[...]
    dinv = dinv_ref[...]
    s = a[0, :, :16] + a[1, :, :16] + p_ref[...][:, :16]
    h = jnp.tanh(dinv * s + b_ref[...])
    o_ref[...] = _pad(dinv * h)


def _t_post2(a_ref, p_ref, dinv_ref, w_ref, b_ref, o_ref):
    a = a_ref[...]
    dinv = dinv_ref[...]
    t = dinv * (a[0, :, :16] + a[1, :, :16] + p_ref[...][:, :16])
    h = jnp.tanh(_dot(t, w_ref[...]) + b_ref[...])
    o_ref[...] = dinv * h


def _t_post3(a_ref, p_ref, dinv_ref, w3_ref, b3_ref, w4_ref, o_ref):
    a = a_ref[...]
    dinv = dinv_ref[...]
    t = dinv * (a[0] + a[1] + p_ref[...])
    h3 = jnp.tanh(_dot(t, w3_ref[...]) + b3_ref[...])
    o_ref[...] = _pad(dinv * _dot(h3, w4_ref[...]))


def _t_post4(a_ref, p_ref, dinv_ref, b_ref, o_ref):
    a = a_ref[...]
    dinv = dinv_ref[...]
    s = a[0, :, :64] + a[1, :, :64] + p_ref[...][:, :64]
    h = jnp.tanh(dinv * s + b_ref[...])
    o_ref[...] = _pad(dinv * h)


def _t_post5(a_ref, p_ref, dinv_ref, w5_ref, b5_ref, batch_ref, wc_ref,
             bc_ref, o_ref):
    a = a_ref[...]
    dinv = dinv_ref[...]
    t = dinv * (a[0, :, :64] + a[1, :, :64] + p_ref[...][:, :64])
    h5 = jnp.tanh(_dot(t, w5_ref[...]) + b5_ref[...])
    seg = batch_ref[...]
    gid = lax.broadcasted_iota(jnp.int32, (_G, _N), 0)
    oh = (gid == seg).astype(jnp.float32)
    sums = _dot(oh, h5)
    cnts = jnp.sum(oh, axis=1, keepdims=True)
    pooled = sums / jnp.maximum(cnts, 1.0)
    o_ref[...] = _dot(pooled, wc_ref[...]) + bc_ref[...]


def _tc(body, out_shape, *args):
    return pl.pallas_call(body, out_shape=out_shape)(*args)


def kernel(x, edge_index, batch, W1, b1, W2, b2, W3, b3, W4, b4, W5, b5,
           Wc, bc):
    f32 = jnp.float32
    src = edge_index[0]
    dst = edge_index[1].reshape(_NC * _NS, _NCHUNK, _C)

    z1 = _tc(_t_xw1, jax.ShapeDtypeStruct((_N, _F), f32), x, W1)
    degp = _deg_call(dst)
    dinv, p1 = _tc(
        _t_dinv_p1,
        [jax.ShapeDtypeStruct((_N, 1), f32), jax.ShapeDtypeStruct((_N, _F), f32)],
        degp, z1)

    a1 = _agg_call(p1, src, dst)
    p2 = _tc(_t_post1, jax.ShapeDtypeStruct((_N, _F), f32),
             a1, p1, dinv, b1.reshape(1, 16))
    a2 = _agg_call(p2, src, dst)
    p3 = _tc(_t_post2, jax.ShapeDtypeStruct((_N, _F), f32),
             a2, p2, dinv, W2, b2.reshape(1, 128))
    a3 = _agg_call(p3, src, dst)
    p4 = _tc(_t_post3, jax.ShapeDtypeStruct((_N, _F), f32),
             a3, p3, dinv, W3, b3.reshape(1, 128), W4)
    a4 = _agg_call(p4, src, dst)
    p5 = _tc(_t_post4, jax.ShapeDtypeStruct((_N, _F), f32),
             a4, p4, dinv, b4.reshape(1, 64))
    a5 = _agg_call(p5, src, dst)
    out = _tc(_t_post5, jax.ShapeDtypeStruct((_G, 1), f32),
              a5, p5, dinv, W5, b5.reshape(1, 64), batch.reshape(1, _N),
              Wc, bc.reshape(1, 1))
    return out


# speedup vs baseline: 21.0756x; 1.0158x over previous
import functools

import jax
import jax.numpy as jnp
from jax import lax
from jax.experimental import pallas as pl
from jax.experimental.pallas import tpu as pltpu
from jax.experimental.pallas import tpu_sc as plsc

_N = 10000
_E = 320000
_G = 64
_F = 128
_NC = 2
_NS = 16
_C = 80
_PER_TILE = _E // (_NC * _NS)
_NCHUNK = _PER_TILE // _C
_RPT = 624
_TAIL0 = _NS * _RPT
_TAIL = _N - _TAIL0
_ZR = 16
_DC = 125
_DNCH = _PER_TILE // _DC
_DAHEAD = 8

_HIGH = jax.lax.Precision.HIGHEST


def _fill(buf, rows, value):
    @pl.loop(0, rows)
    def _(r):
        @pl.loop(0, _F, step=16)
        def _(c0):
            buf[r, pl.ds(c0, 16)] = jnp.full((16,), value, jnp.float32)


def _zero_acc(acc, zbuf, sid):
    row0 = sid * _RPT
    off = 0
    while off < _RPT:
        nr = min(_ZR, _RPT - off)
        pltpu.sync_copy(zbuf.at[pl.ds(0, nr)], acc.at[pl.ds(row0 + off, nr)])
        off += nr

    @pl.when(sid == 0)
    def _():
        pltpu.sync_copy(zbuf.at[pl.ds(0, _TAIL)], acc.at[pl.ds(_TAIL0, _TAIL)])

    return row0


def _copy_out(acc, out_hbm, cid, sid, row0):
    pltpu.sync_copy(acc.at[pl.ds(row0, _RPT)],
                    out_hbm.at[cid, pl.ds(row0, _RPT)])

    @pl.when(sid == 0)
    def _():
        pltpu.sync_copy(acc.at[pl.ds(_TAIL0, _TAIL)],
                        out_hbm.at[cid, pl.ds(_TAIL0, _TAIL)])


def _make_agg():
    mesh = plsc.VectorSubcoreMesh(core_axis_name="c", subcore_axis_name="s")

    @functools.partial(
        pl.kernel, mesh=mesh,
        out_type=jax.ShapeDtypeStruct((_NC, _N, _F), jnp.float32),
        scratch_types=[
            pltpu.VMEM((_PER_TILE,), jnp.int32),
            pltpu.VMEM((_NCHUNK, _C), jnp.int32),
            pltpu.VMEM((_C, _F), jnp.float32),
            pltpu.VMEM((_C, _F), jnp.float32),
            pltpu.VMEM((_ZR, _F), jnp.float32),
            pltpu.VMEM_SHARED((_N, _F), jnp.float32),
            pltpu.SemaphoreType.DMA,
            pltpu.SemaphoreType.DMA,
        ],
    )
    def k(p_hbm, src_hbm, dst_hbm, out_hbm, src_v, dst_v, bufa, bufb, zbuf,
          acc, sema, semb):
        cid = lax.axis_index("c")
        sid = lax.axis_index("s")
        tile = cid * _NS + sid
        pltpu.sync_copy(src_hbm.at[pl.ds(tile * _PER_TILE, _PER_TILE)], src_v)
        pltpu.sync_copy(dst_hbm.at[tile], dst_v)
        _fill(zbuf, _ZR, 0.0)
        row0 = _zero_acc(acc, zbuf, sid)
        plsc.subcore_barrier()

        def issue(i, buf, sem):
            @pl.when(i < _NCHUNK)
            def _():
                pltpu.async_copy(p_hbm.at[src_v.at[pl.ds(i * _C, _C)]],
                                 buf, sem)

        def wait(buf, sem):
            pltpu.make_async_copy(p_hbm.at[src_v.at[pl.ds(0, _C)]],
                                  buf, sem).wait()

        issue(0, bufa, sema)
        issue(1, bufb, semb)

        @pl.loop(0, (_NCHUNK + 1) // 2)
        def _(j):
            i0 = 2 * j
            wait(bufa, sema)
            pltpu.sync_copy(bufa, acc.at[dst_v.at[i0]], add=True)
            issue(i0 + 2, bufa, sema)

            @pl.when(i0 + 1 < _NCHUNK)
            def _():
                wait(bufb, semb)
                pltpu.sync_copy(bufb, acc.at[dst_v.at[i0 + 1]], add=True)
                issue(i0 + 3, bufb, semb)

        plsc.subcore_barrier()
        _copy_out(acc, out_hbm, cid, sid, row0)

    return k


def _make_deg():
    mesh = plsc.VectorSubcoreMesh(core_axis_name="c", subcore_axis_name="s")

    @functools.partial(
        pl.kernel, mesh=mesh,
        out_type=jax.ShapeDtypeStruct((_NC, _N, _F), jnp.float32),
        scratch_types=[
            pltpu.VMEM((_DNCH, _DC), jnp.int32),
            pltpu.VMEM((_DC, _F), jnp.float32),
            pltpu.VMEM((_ZR, _F), jnp.float32),
            pltpu.VMEM_SHARED((_N, _F), jnp.float32),
            pltpu.SemaphoreType.DMA,
        ],
    )
    def k(dst_hbm, out_hbm, dst_v, ones_v, zbuf, acc, sem):
        cid = lax.axis_index("c")
        sid = lax.axis_index("s")
        tile = cid * _NS + sid
        pltpu.sync_copy(dst_hbm.at[tile], dst_v)
        _fill(ones_v, _DC, 1.0)
        _fill(zbuf, _ZR, 0.0)
        row0 = _zero_acc(acc, zbuf, sid)
        plsc.subcore_barrier()

        def issue(i):
            @pl.when(i < _DNCH)
            def _():
                pltpu.async_copy(ones_v, acc.at[dst_v.at[i]], sem, add=True)

        for i in range(_DAHEAD):
            issue(i)

        @pl.loop(0, _DNCH)
        def _(i):
            pltpu.make_async_copy(ones_v, acc.at[dst_v.at[0]], sem).wait()
            issue(i + _DAHEAD)

        plsc.subcore_barrier()
        _copy_out(acc, out_hbm, cid, sid, row0)

    return k


_deg_call = _make_deg()
_agg_call = _make_agg()


def _dot(a, b):
    return jnp.dot(a, b, preferred_element_type=jnp.float32, precision=_HIGH)


def _pad(v):
    n, f = v.shape
    if f == _F:
        return v
    return jnp.concatenate([v, jnp.zeros((n, _F - f), jnp.float32)], axis=1)


def _t_xw1(x_ref, w_ref, o_ref):
    o_ref[...] = _pad(_dot(x_ref[...], w_ref[...]))


def _t_dinv_p1(degp_ref, z_ref, dinv_ref, p_ref):
    d = degp_ref[...]
    deg = d[0, :, 0:1] + d[1, :, 0:1] + 1.0
    dinv = jax.lax.rsqrt(deg)
    dinv_ref[...] = dinv
    p_ref[...] = z_ref[...] * dinv


def _t_post1(a_ref, p_ref, dinv_ref, b_ref, o_ref):
    a = a_ref[...]
    dinv = dinv_ref[...]
    s = a[0, :, :16] + a[1, :, :16] + p_ref[...][:, :16]
    h = jnp.tanh(dinv * s + b_ref[...])
    o_ref[...] = _pad(dinv * h)


def _t_post2(a_ref, p_ref, dinv_ref, w_ref, b_ref, o_ref):
    a = a_ref[...]
    dinv = dinv_ref[...]
    t = dinv * (a[0, :, :16] + a[1, :, :16] + p_ref[...][:, :16])
    h = jnp.tanh(_dot(t, w_ref[...]) + b_ref[...])
    o_ref[...] = dinv * h


def _t_post3(a_ref, p_ref, dinv_ref, w3_ref, b3_ref, w4_ref, o_ref):
    a = a_ref[...]
    dinv = dinv_ref[...]
    t = dinv * (a[0] + a[1] + p_ref[...])
    h3 = jnp.tanh(_dot(t, w3_ref[...]) + b3_ref[...])
    o_ref[...] = _pad(dinv * _dot(h3, w4_ref[...]))


def _t_post4(a_ref, p_ref, dinv_ref, b_ref, o_ref):
    a = a_ref[...]
    dinv = dinv_ref[...]
    s = a[0, :, :64] + a[1, :, :64] + p_ref[...][:, :64]
    h = jnp.tanh(dinv * s + b_ref[...])
    o_ref[...] = _pad(dinv * h)


def _t_post5(a_ref, p_ref, dinv_ref, w5_ref, b5_ref, batch_ref, wc_ref,
             bc_ref, o_ref):
    a = a_ref[...]
    dinv = dinv_ref[...]
    t = dinv * (a[0, :, :64] + a[1, :, :64] + p_ref[...][:, :64])
    h5 = jnp.tanh(_dot(t, w5_ref[...]) + b5_ref[...])
    seg = batch_ref[...]
    gid = lax.broadcasted_iota(jnp.int32, (_G, _N), 0)
    oh = (gid == seg).astype(jnp.float32)
    sums = _dot(oh, h5)
    cnts = jnp.sum(oh, axis=1, keepdims=True)
    pooled = sums / jnp.maximum(cnts, 1.0)
    o_ref[...] = _dot(pooled, wc_ref[...]) + bc_ref[...]


def _tc(body, out_shape, *args):
    return pl.pallas_call(body, out_shape=out_shape)(*args)


def kernel(x, edge_index, batch, W1, b1, W2, b2, W3, b3, W4, b4, W5, b5,
           Wc, bc):
    f32 = jnp.float32
    srcf = edge_index[0]
    dst = edge_index[1].reshape(_NC * _NS, _NCHUNK, _C)
    dstd = edge_index[1].reshape(_NC * _NS, _DNCH, _DC)

    z1 = _tc(_t_xw1, jax.ShapeDtypeStruct((_N, _F), f32), x, W1)
    degp = _deg_call(dstd)
    dinv, p1 = _tc(
        _t_dinv_p1,
        [jax.ShapeDtypeStruct((_N, 1), f32), jax.ShapeDtypeStruct((_N, _F), f32)],
        degp, z1)

    a1 = _agg_call(p1, srcf, dst)
    p2 = _tc(_t_post1, jax.ShapeDtypeStruct((_N, _F), f32),
             a1, p1, dinv, b1.reshape(1, 16))
    a2 = _agg_call(p2, srcf, dst)
    p3 = _tc(_t_post2, jax.ShapeDtypeStruct((_N, _F), f32),
             a2, p2, dinv, W2, b2.reshape(1, 128))
    a3 = _agg_call(p3, srcf, dst)
    p4 = _tc(_t_post3, jax.ShapeDtypeStruct((_N, _F), f32),
             a3, p3, dinv, W3, b3.reshape(1, 128), W4)
    a4 = _agg_call(p4, srcf, dst)
    p5 = _tc(_t_post4, jax.ShapeDtypeStruct((_N, _F), f32),
             a4, p4, dinv, b4.reshape(1, 64))
    a5 = _agg_call(p5, srcf, dst)
    out = _tc(_t_post5, jax.ShapeDtypeStruct((_G, 1), f32),
              a5, p5, dinv, W5, b5.reshape(1, 64), batch.reshape(1, _N),
              Wc, bc.reshape(1, 1))
    return out
